# baseline (speedup 1.0000x reference)
import functools
import math

import jax
import jax.numpy as jnp
from jax import lax
from jax.experimental import pallas as pl
from jax.experimental.pallas import tpu as pltpu
from jax.experimental.pallas import tpu_sc as plsc

_NROW = 4096
_NCOL = 200
_D = 32
_NW = 32
_BW = _NROW // _NW
_GT = _D // 8
_SCALE = math.sqrt(float(_D))

_mesh = plsc.VectorSubcoreMesh(core_axis_name="c", subcore_axis_name="s")


@functools.partial(
    pl.kernel,
    out_type=jax.ShapeDtypeStruct((_NCOL, _GT, _NW, 1024), jnp.float32),
    mesh=_mesh,
    compiler_params=pltpu.CompilerParams(
        use_tc_tiling_on_sc=False, needs_layout_passes=False
    ),
    scratch_types=[
        pltpu.VMEM((_NCOL, _BW), jnp.int32),
        pltpu.VMEM((2, _BW, _D), jnp.float32),
        pltpu.VMEM((2, _GT * 1024), jnp.float32),
        pltpu.SemaphoreType.DMA,
        pltpu.SemaphoreType.DMA,
        pltpu.SemaphoreType.DMA,
        pltpu.SemaphoreType.DMA,
    ],
)
def _emb_lookup(tokens_hbm, table_hbm, out_hbm, idx_v, rows_v, tr_v,
                gsem0, gsem1, wsem0, wsem1):
    wid = lax.axis_index("s") * 2 + lax.axis_index("c")
    gsems = (gsem0, gsem1)
    wsems = (wsem0, wsem1)
    pltpu.sync_copy(tokens_hbm.at[:, pl.ds(wid * _BW, _BW)], idx_v)

    lane = jax.lax.iota(jnp.int32, 16)
    half_base = [(lane + (16 * h)) * 128 for h in range(2)]

    def gather_desc(j, b):
        return pltpu.make_async_copy(
            table_hbm.at[idx_v.at[j]], rows_v.at[b], gsems[b]
        )

    def write_descs(j, b):
        return [
            pltpu.make_async_copy(
                tr_v.at[b, pl.ds(g * 1024, 1024)],
                out_hbm.at[j, g, wid],
                wsems[b],
            )
            for g in range(_GT)
        ]

    def step(j, b, fire_next, wait_prev_write):
        gather_desc(j, b).wait()
        if wait_prev_write:
            for d in write_descs(j - 2, b):
                d.wait()
        buf = rows_v.at[b]
        dst = tr_v.at[b]
        for r in range(_BW):
            for h in range(2):
                v = buf[r, pl.ds(16 * h, 16)]
                plsc.store_scatter(dst, [half_base[h] + r], v * _SCALE)
        if fire_next:
            gather_desc(j + 2, b).start()
        for d in write_descs(j, b):
            d.start()

    gather_desc(0, 0).start()
    gather_desc(1, 1).start()
    step(0, 0, True, False)
    step(1, 1, True, False)

    def body(jj, carry):
        j = 2 * jj
        step(j, 0, True, True)
        step(j + 1, 1, True, True)
        return carry

    lax.fori_loop(1, _NCOL // 2 - 1, body, 0)

    step(_NCOL - 2, 0, False, True)
    step(_NCOL - 1, 1, False, True)
    for d in write_descs(_NCOL - 2, 0):
        d.wait()
    for d in write_descs(_NCOL - 1, 1):
        d.wait()


def kernel(tokens, table):
    kout = _emb_lookup(tokens.T.astype(jnp.int32), table)
    k5 = kout.reshape(_NCOL, _GT, _NW, 8, 128)
    return k5.transpose(2, 4, 0, 1, 3).reshape(_NROW, _NCOL, _D)


# speedup vs baseline: 1.3020x; 1.3020x over previous
import functools
import math

import jax
import jax.numpy as jnp
from jax import lax
from jax.experimental import pallas as pl
from jax.experimental.pallas import tpu as pltpu
from jax.experimental.pallas import tpu_sc as plsc

_NROW = 4096
_NCOL = 200
_D = 32
_NW = 32
_BW = _NROW // _NW
_GT = _D // 8
_SCALE = math.sqrt(float(_D))

_mesh = plsc.VectorSubcoreMesh(core_axis_name="c", subcore_axis_name="s")


@functools.partial(
    pl.kernel,
    out_type=jax.ShapeDtypeStruct((_NCOL, _GT, _NW, 8, 128), jnp.float32),
    mesh=_mesh,
    compiler_params=pltpu.CompilerParams(
        use_tc_tiling_on_sc=False, needs_layout_passes=False
    ),
    scratch_types=[
        pltpu.VMEM((_NCOL, _BW), jnp.int32),
        pltpu.VMEM((2, _BW, _D), jnp.float32),
        pltpu.VMEM((2, _D, 129), jnp.float32),
        pltpu.SemaphoreType.DMA,
        pltpu.SemaphoreType.DMA,
        pltpu.SemaphoreType.DMA,
        pltpu.SemaphoreType.DMA,
    ],
)
def _emb_lookup(tokens_hbm, table_hbm, out_hbm, idx_v, rows_v, tr_v,
                gsem0, gsem1, wsem0, wsem1):
    wid = lax.axis_index("s") * 2 + lax.axis_index("c")
    gsems = (gsem0, gsem1)
    wsems = (wsem0, wsem1)
    pltpu.sync_copy(tokens_hbm.at[:, pl.ds(wid * _BW, _BW)], idx_v)

    lane = jax.lax.iota(jnp.int32, 16)
    half_dim = [lane + (16 * h) for h in range(2)]

    def gather_desc(j, b):
        return pltpu.make_async_copy(
            table_hbm.at[idx_v.at[j]], rows_v.at[b], gsems[b]
        )

    def write_descs(j, b):
        return [
            pltpu.make_async_copy(
                tr_v.at[b, pl.ds(g * 8, 8), pl.ds(0, 128)],
                out_hbm.at[j, g, wid],
                wsems[b],
            )
            for g in range(_GT)
        ]

    def step(j, b, fire_next, wait_prev_write):
        gather_desc(j, b).wait()
        if wait_prev_write:
            for d in write_descs(j - 2, b):
                d.wait()
        buf = rows_v.at[b]
        dst = tr_v.at[b]
        for r in range(_BW):
            rv = jnp.full((16,), r, dtype=jnp.int32)
            for h in range(2):
                v = buf[r, pl.ds(16 * h, 16)]
                plsc.store_scatter(dst, [half_dim[h], rv], v * _SCALE)
        if fire_next:
            gather_desc(j + 2, b).start()
        for d in write_descs(j, b):
            d.start()

    gather_desc(0, 0).start()
    gather_desc(1, 1).start()
    step(0, 0, True, False)
    step(1, 1, True, False)

    def body(jj, carry):
        j = 2 * jj
        step(j, 0, True, True)
        step(j + 1, 1, True, True)
        return carry

    lax.fori_loop(1, _NCOL // 2 - 1, body, 0)

    step(_NCOL - 2, 0, False, True)
    step(_NCOL - 1, 1, False, True)
    for d in write_descs(_NCOL - 2, 0):
        d.wait()
    for d in write_descs(_NCOL - 1, 1):
        d.wait()


def kernel(tokens, table):
    kout = _emb_lookup(tokens.T.astype(jnp.int32), table)
    return kout.transpose(2, 4, 0, 1, 3).reshape(_NROW, _NCOL, _D)


# speedup vs baseline: 1.3051x; 1.0024x over previous
import functools
import math

import jax
import jax.numpy as jnp
from jax import lax
from jax.experimental import pallas as pl
from jax.experimental.pallas import tpu as pltpu
from jax.experimental.pallas import tpu_sc as plsc

_NROW = 4096
_NCOL = 200
_D = 32
_NW = 32
_BW = _NROW // _NW
_GT = _D // 8
_SCALE = math.sqrt(float(_D))

_mesh = plsc.VectorSubcoreMesh(core_axis_name="c", subcore_axis_name="s")


@functools.partial(
    pl.kernel,
    out_type=jax.ShapeDtypeStruct((_NCOL, _GT, _NW, 8, 128), jnp.float32),
    mesh=_mesh,
    compiler_params=pltpu.CompilerParams(
        use_tc_tiling_on_sc=False, needs_layout_passes=False
    ),
    scratch_types=[
        pltpu.VMEM((25, 8, _BW), jnp.int32),
        pltpu.VMEM((2, _BW, _D), jnp.float32),
        pltpu.VMEM((2, _D, 129), jnp.float32),
        pltpu.SemaphoreType.DMA,
        pltpu.SemaphoreType.DMA,
        pltpu.SemaphoreType.DMA,
        pltpu.SemaphoreType.DMA,
    ],
)
def _emb_lookup(tokens_hbm, table_hbm, out_hbm, idx_v, rows_v, tr_v,
                gsem0, gsem1, wsem0, wsem1):
    wid = lax.axis_index("s") * 2 + lax.axis_index("c")
    gsems = (gsem0, gsem1)
    wsems = (wsem0, wsem1)
    pltpu.sync_copy(tokens_hbm.at[:, wid], idx_v)

    lane = jax.lax.iota(jnp.int32, 16)
    half_dim = [lane + (16 * h) for h in range(2)]

    def gather_desc(j, b):
        return pltpu.make_async_copy(
            table_hbm.at[idx_v.at[j // 8, j % 8]], rows_v.at[b], gsems[b]
        )

    def write_descs(j, b):
        return [
            pltpu.make_async_copy(
                tr_v.at[b, pl.ds(g * 8, 8), pl.ds(0, 128)],
                out_hbm.at[j, g, wid],
                wsems[b],
            )
            for g in range(_GT)
        ]

    def step(j, b, fire_next, wait_prev_write):
        gather_desc(j, b).wait()
        if wait_prev_write:
            for d in write_descs(j - 2, b):
                d.wait()
        buf = rows_v.at[b]
        dst = tr_v.at[b]
        for r in range(_BW):
            rv = jnp.full((16,), r, dtype=jnp.int32)
            for h in range(2):
                v = buf[r, pl.ds(16 * h, 16)]
                plsc.store_scatter(dst, [half_dim[h], rv], v * _SCALE)
        if fire_next:
            gather_desc(j + 2, b).start()
        for d in write_descs(j, b):
            d.start()

    gather_desc(0, 0).start()
    gather_desc(1, 1).start()
    step(0, 0, True, False)
    step(1, 1, True, False)

    def body(jj, carry):
        j = 2 * jj
        step(j, 0, True, True)
        step(j + 1, 1, True, True)
        return carry

    lax.fori_loop(1, _NCOL // 2 - 1, body, 0)

    step(_NCOL - 2, 0, False, True)
    step(_NCOL - 1, 1, False, True)
    for d in write_descs(_NCOL - 2, 0):
        d.wait()
    for d in write_descs(_NCOL - 1, 1):
        d.wait()


def kernel(tokens, table):
    tok4 = tokens.T.astype(jnp.int32).reshape(25, 8, _NW, 128)
    kout = _emb_lookup(tok4.transpose(0, 2, 1, 3), table)
    return kout.transpose(2, 4, 0, 1, 3).reshape(_NROW, _NCOL, _D)


# speedup vs baseline: 1.8533x; 1.4201x over previous
import functools
import math

import jax
import jax.numpy as jnp
from jax import lax
from jax.experimental import pallas as pl
from jax.experimental.pallas import tpu as pltpu
from jax.experimental.pallas import tpu_sc as plsc

_NROW = 4096
_NCOL = 200
_D = 32
_NW = 32
_BW = _NROW // _NW
_GT = _D // 8
_NG = _NCOL // 8
_SCALE = math.sqrt(float(_D))

_mesh = plsc.VectorSubcoreMesh(core_axis_name="c", subcore_axis_name="s")


@functools.partial(
    pl.kernel,
    out_type=jax.ShapeDtypeStruct((_NCOL, _GT, _NW, 8, 128), jnp.float32),
    mesh=_mesh,
    compiler_params=pltpu.CompilerParams(
        use_tc_tiling_on_sc=False, needs_layout_passes=False
    ),
    scratch_types=[
        pltpu.VMEM((_NG, 8, _BW), jnp.int32),
        pltpu.VMEM((4, _BW, _D), jnp.float32),
        pltpu.VMEM((4, _D, 129), jnp.float32),
    ] + [pltpu.SemaphoreType.DMA] * 8,
)
def _emb_lookup(tokens_hbm, table_hbm, out_hbm, idx_v, rows_v, tr_v,
                *sems):
    gsem = sems[:4]
    wsem = sems[4:]
    wid = lax.axis_index("s") * 2 + lax.axis_index("c")
    pltpu.sync_copy(tokens_hbm.at[:, wid], idx_v)

    lane = jax.lax.iota(jnp.int32, 16)
    half_dim = [lane + (16 * h) for h in range(2)]

    def gather_desc(t, jc, s):
        return pltpu.make_async_copy(
            table_hbm.at[idx_v.at[t // 2, 4 * (t % 2) + jc]],
            rows_v.at[s],
            gsem[s],
        )

    def write_descs(t, jc):
        return [
            pltpu.make_async_copy(
                tr_v.at[jc, pl.ds(g * 8, 8), pl.ds(0, 128)],
                out_hbm.at[4 * t + jc, g, wid],
                wsem[jc],
            )
            for g in range(_GT)
        ]

    def do_group(t, fire_next, wait_prev_write):
        for jc in range(4):
            gather_desc(t, jc, jc).wait()
            if wait_prev_write:
                for d in write_descs(t - 1, jc):
                    d.wait()
            buf = rows_v.at[jc]
            dst = tr_v.at[jc]

            @plsc.parallel_loop(0, _BW, unroll=8)
            def _(r):
                rv = jax.lax.broadcast(r, (16,))
                for h in range(2):
                    v = buf[r, pl.ds(16 * h, 16)]
                    plsc.store_scatter(dst, [half_dim[h], rv], v * _SCALE)
            if fire_next:
                gather_desc(t + 1, jc, jc).start()
            for d in write_descs(t, jc):
                d.start()

    for jc in range(4):
        gather_desc(0, jc, jc).start()
    do_group(0, True, False)

    def body(t, carry):
        do_group(t, True, True)
        return carry

    lax.fori_loop(1, 2 * _NG - 1, body, 0)

    do_group(2 * _NG - 1, False, True)
    for jc in range(4):
        for d in write_descs(2 * _NG - 1, jc):
            d.wait()


def kernel(tokens, table):
    tok4 = tokens.T.astype(jnp.int32).reshape(_NG, 8, _NW, 128)
    kout = _emb_lookup(tok4.transpose(0, 2, 1, 3), table)
    return kout.transpose(2, 4, 0, 1, 3).reshape(_NROW, _NCOL, _D)


# speedup vs baseline: 1.8669x; 1.0073x over previous
import functools
import math

import jax
import jax.numpy as jnp
from jax import lax
from jax.experimental import pallas as pl
from jax.experimental.pallas import tpu as pltpu
from jax.experimental.pallas import tpu_sc as plsc

_NROW = 4096
_NCOL = 200
_D = 32
_NW = 32
_BW = _NROW // _NW
_GT = _D // 8
_NG = _NCOL // 8
_SCALE = math.sqrt(float(_D))

_mesh = plsc.VectorSubcoreMesh(core_axis_name="c", subcore_axis_name="s")


@functools.partial(
    pl.kernel,
    out_type=jax.ShapeDtypeStruct((_NCOL, _GT, _NW, 8, 128), jnp.float32),
    mesh=_mesh,
    compiler_params=pltpu.CompilerParams(
        use_tc_tiling_on_sc=False, needs_layout_passes=False
    ),
    scratch_types=[
        pltpu.VMEM((_NG, 8, _BW), jnp.int32),
        pltpu.VMEM((8, _BW, _D), jnp.float32),
        pltpu.VMEM((8, _D, 129), jnp.float32),
    ] + [pltpu.SemaphoreType.DMA] * 16,
)
def _emb_lookup(tokens_hbm, table_hbm, out_hbm, idx_v, rows_v, tr_v,
                *sems):
    gsem = sems[:8]
    wsem = sems[8:]
    wid = lax.axis_index("s") * 2 + lax.axis_index("c")
    pltpu.sync_copy(tokens_hbm.at[:, wid], idx_v)

    lane = jax.lax.iota(jnp.int32, 16)
    half_dim = [lane + (16 * h) for h in range(2)]

    def gather_desc(t, jc, s):
        return pltpu.make_async_copy(
            table_hbm.at[idx_v.at[t, jc]],
            rows_v.at[s],
            gsem[s],
        )

    def write_descs(t, jc):
        return [
            pltpu.make_async_copy(
                tr_v.at[jc, pl.ds(g * 8, 8), pl.ds(0, 128)],
                out_hbm.at[8 * t + jc, g, wid],
                wsem[jc],
            )
            for g in range(_GT)
        ]

    def do_group(t, fire_next, wait_prev_write):
        for jc in range(8):
            gather_desc(t, jc, jc).wait()
            if wait_prev_write:
                for d in write_descs(t - 1, jc):
                    d.wait()
            buf = rows_v.at[jc]
            dst = tr_v.at[jc]

            @plsc.parallel_loop(0, _BW, unroll=8)
            def _(r):
                rv = jax.lax.broadcast(r, (16,))
                for h in range(2):
                    v = buf[r, pl.ds(16 * h, 16)]
                    plsc.store_scatter(dst, [half_dim[h], rv], v * _SCALE)
            if fire_next:
                gather_desc(t + 1, jc, jc).start()
            for d in write_descs(t, jc):
                d.start()

    for jc in range(8):
        gather_desc(0, jc, jc).start()
    do_group(0, True, False)

    def body(t, carry):
        do_group(t, True, True)
        return carry

    lax.fori_loop(1, _NG - 1, body, 0)

    do_group(_NG - 1, False, True)
    for jc in range(8):
        for d in write_descs(_NG - 1, jc):
            d.wait()


def kernel(tokens, table):
    tok4 = tokens.T.astype(jnp.int32).reshape(_NG, 8, _NW, 128)
    kout = _emb_lookup(tok4.transpose(0, 2, 1, 3), table)
    return kout.transpose(2, 4, 0, 1, 3).reshape(_NROW, _NCOL, _D)
